# baseline (speedup 1.0000x reference)
import functools

import jax
import jax.numpy as jnp
from jax import lax
from jax.experimental import pallas as pl
from jax.experimental.pallas import tpu as pltpu
from jax.experimental.pallas import tpu_sc as plsc

B = 16
T = 16384
D = 128
NR = 5
NR_PAD = 8
IDX_CHUNK = 128
N_CHUNKS = 1
TC_CHUNK = T // N_CHUNKS
TBLK = 4096


def _sc_gather(chunk, flat_users, nodes_v, embed_u_w, embed_i_w):
    info = plsc.get_sparse_core_info()
    nc, ns = info.num_cores, info.num_subcores
    nw = nc * ns
    rows_per_w = TC_CHUNK // nw
    n_idx_chunks = max(rows_per_w // IDX_CHUNK, 1)
    chunk_base = chunk * TC_CHUNK
    with_qj = chunk == 0

    mesh = plsc.VectorSubcoreMesh(core_axis_name="c", subcore_axis_name="s")

    out_type = [jax.ShapeDtypeStruct((TC_CHUNK, D), jnp.float32)]
    scratch = [
        pltpu.VMEM((rows_per_w,), jnp.int32),
        pltpu.VMEM((rows_per_w, D), jnp.float32),
        pltpu.SemaphoreType.DMA,
    ]
    if with_qj:
        out_type.append(jax.ShapeDtypeStruct((B, D), jnp.float32))
        scratch += [
            pltpu.VMEM((B,), jnp.int32),
            pltpu.VMEM((B, D), jnp.float32),
            pltpu.SemaphoreType.DMA,
        ]

    @functools.partial(pl.kernel, mesh=mesh, out_type=out_type,
                       scratch_types=scratch)
    def gather_kernel(*refs):
        if with_qj:
            (users_hbm, nodes_hbm, tab_u, tab_i, out_pt, out_qj,
             idx_v, rows_v, sem, nidx_v, qrows_v, qsem) = refs
        else:
            users_hbm, tab_u, out_pt, idx_v, rows_v, sem = refs
        wid = lax.axis_index("s") * nc + lax.axis_index("c")
        base = wid * rows_per_w
        pltpu.sync_copy(users_hbm.at[pl.ds(chunk_base + base, rows_per_w)], idx_v)
        copies = []
        for c in range(n_idx_chunks):
            copies.append(pltpu.async_copy(
                tab_u.at[idx_v.at[pl.ds(c * IDX_CHUNK, IDX_CHUNK)]],
                rows_v.at[pl.ds(c * IDX_CHUNK, IDX_CHUNK)], sem))
        for cp in copies:
            cp.wait()
        pltpu.sync_copy(rows_v, out_pt.at[pl.ds(base, rows_per_w)])

        if with_qj:
            @pl.when(wid == 0)
            def _():
                pltpu.sync_copy(nodes_hbm, nidx_v)
                pltpu.async_copy(tab_i.at[nidx_v], qrows_v, qsem).wait()
                pltpu.sync_copy(qrows_v, out_qj)

    if with_qj:
        return gather_kernel(flat_users, nodes_v, embed_u_w, embed_i_w)
    return gather_kernel(flat_users, embed_u_w)[0]


_NT = (((1,), (1,)), ((), ()))
_TN = (((0,), (0,)), ((), ()))
_BF = jnp.bfloat16


def _nt_bf16(x, w):
    return lax.dot_general(x.astype(_BF), w.astype(_BF), _NT,
                           preferred_element_type=jnp.float32)


def _tc_body(rat_ref, seg_ref, pt_ref, qj_ref, er_ref, g1_ref, g1b_ref,
             g2_ref, g2b_ref, a1_ref, a1b_ref, a2_ref, a2b_ref, a3_ref,
             *rest, nblk, first, last):
    n_outs = 1 if last else 3
    if first:
        in_state = ()
    else:
        in_state = rest[:3]
        rest = rest[3:]
    outs = rest[:n_outs]
    m_ref, d_ref, z_ref, rtab_ref, stab_ref = rest[n_outs:]

    i = pl.program_id(0)

    @pl.when(i == 0)
    def _init():
        if first:
            m_ref[...] = jnp.full((B, 1), -1e30, jnp.float32)
            d_ref[...] = jnp.zeros((B, 1), jnp.float32)
            z_ref[...] = jnp.zeros((B, D), jnp.float32)
        else:
            m_in, d_in, z_in = in_state
            m_ref[...] = m_in[...]
            d_ref[...] = d_in[...]
            z_ref[...] = z_in[...]
        rtab_ref[:NR, :] = lax.dot_general(er_ref[...], g1_ref[:, D:], _NT)
        rtab_ref[NR:, :] = jnp.zeros((NR_PAD - NR, D), jnp.float32)
        stab_ref[...] = lax.dot_general(qj_ref[...], a1_ref[:, D:], _NT)

    pt = pt_ref[...]
    rat_row = rat_ref[...].reshape(1, TBLK)
    seg_row = seg_ref[...].reshape(1, TBLK)

    rat_ohT = (lax.broadcasted_iota(jnp.int32, (NR_PAD, TBLK), 0)
               == rat_row).astype(jnp.float32)
    seg_ohT = lax.broadcasted_iota(jnp.int32, (B, TBLK), 0) == seg_row

    er_c = lax.dot_general(rat_ohT, rtab_ref[...], _TN)
    h = jnp.maximum(_nt_bf16(pt, g1_ref[:, :D]) + er_c + g1b_ref[...], 0.0)
    fjt = jnp.maximum(_nt_bf16(h, g2_ref[...]) + g2b_ref[...], 0.0)
    seg_c = lax.dot_general(seg_ohT.astype(jnp.float32), stab_ref[...], _TN)
    a = jnp.maximum(_nt_bf16(fjt, a1_ref[:, :D]) + seg_c + a1b_ref[...], 0.0)
    a = jnp.maximum(_nt_bf16(a, a2_ref[...]) + a2b_ref[...], 0.0)
    s_row = lax.dot_general(a3_ref[...], a, _NT)

    neg = jnp.float32(-1e30)
    masked = jnp.where(seg_ohT, s_row, neg)
    blk_m = jnp.max(masked, axis=1, keepdims=True)
    m_old = m_ref[...]
    m_new = jnp.maximum(m_old, blk_m)
    scale = jnp.exp(m_old - m_new)
    e_t = jnp.exp(jnp.where(seg_ohT, s_row - m_new, neg))
    m_ref[...] = m_new
    d_ref[...] = d_ref[...] * scale + jnp.sum(e_t, axis=1, keepdims=True)
    z_ref[...] = (z_ref[...] * scale
                  + lax.dot_general(e_t, fjt, (((1,), (0,)), ((), ()))))

    @pl.when(i == nblk - 1)
    def _finish():
        if last:
            out_ref, = outs
            dd = d_ref[...]
            out_ref[...] = jnp.where(dd > 0, z_ref[...] / dd, 0.0)
        else:
            om_ref, od_ref, oz_ref = outs
            om_ref[...] = m_ref[...]
            od_ref[...] = d_ref[...]
            oz_ref[...] = z_ref[...]


def _tc_chunk(chunk, pt, qj, flat_ratings, segment_ids, embed_r_w, weights,
              state, last):
    (g1_w, g1_b, g2_w, g2_b, a1_w, a1_b, a2_w, a2_b, a3_w) = weights
    nblk = TC_CHUNK // TBLK
    blk0 = chunk * nblk
    first = state is None

    full = lambda shape: pl.BlockSpec(shape, lambda i: tuple(0 for _ in shape))

    in_specs = [
        pl.BlockSpec((TBLK,), lambda i: (blk0 + i,)),
        pl.BlockSpec((TBLK,), lambda i: (blk0 + i,)),
        pl.BlockSpec((TBLK, D), lambda i: (i, 0)),
        full((B, D)),
        full((NR, D)),
        full((D, 2 * D)),
        full((D,)),
        full((D, D)),
        full((D,)),
        full((D, 2 * D)),
        full((D,)),
        full((D, D)),
        full((D,)),
        full((1, D)),
    ]
    args = [flat_ratings, segment_ids, pt, qj, embed_r_w, g1_w, g1_b, g2_w,
            g2_b, a1_w, a1_b, a2_w, a2_b, a3_w]
    if not first:
        in_specs += [full((B, 1)), full((B, 1)), full((B, D))]
        args += list(state)

    if last:
        out_shape = jax.ShapeDtypeStruct((B, D), jnp.float32)
        out_specs = pl.BlockSpec((B, D), lambda i: (0, 0))
    else:
        out_shape = [jax.ShapeDtypeStruct((B, 1), jnp.float32),
                     jax.ShapeDtypeStruct((B, 1), jnp.float32),
                     jax.ShapeDtypeStruct((B, D), jnp.float32)]
        out_specs = [pl.BlockSpec((B, 1), lambda i: (0, 0)),
                     pl.BlockSpec((B, 1), lambda i: (0, 0)),
                     pl.BlockSpec((B, D), lambda i: (0, 0))]

    body = functools.partial(_tc_body, nblk=nblk, first=first, last=last)
    return pl.pallas_call(
        body,
        grid=(nblk,),
        in_specs=in_specs,
        out_specs=out_specs,
        out_shape=out_shape,
        scratch_shapes=[
            pltpu.VMEM((B, 1), jnp.float32),
            pltpu.VMEM((B, 1), jnp.float32),
            pltpu.VMEM((B, D), jnp.float32),
            pltpu.VMEM((NR_PAD, D), jnp.float32),
            pltpu.VMEM((B, D), jnp.float32),
        ],
    )(*args)


def kernel(nodes_v, flat_users, flat_ratings, segment_ids, embed_u_w,
           embed_i_w, embed_r_w, g1_w, g1_b, g2_w, g2_b, a1_w, a1_b,
           a2_w, a2_b, a3_w, a3_b):
    weights = (g1_w, g1_b, g2_w, g2_b, a1_w, a1_b, a2_w, a2_b, a3_w)

    pt0, qj = _sc_gather(0, flat_users, nodes_v, embed_u_w, embed_i_w)
    pts = [pt0] + [_sc_gather(c, flat_users, nodes_v, embed_u_w, embed_i_w)
                   for c in range(1, N_CHUNKS)]

    state = None
    for c in range(N_CHUNKS):
        out = _tc_chunk(c, pts[c], qj, flat_ratings, segment_ids, embed_r_w,
                        weights, state, last=(c == N_CHUNKS - 1))
        state = out
    return out


# speedup vs baseline: 1.0206x; 1.0206x over previous
import functools

import jax
import jax.numpy as jnp
from jax import lax
from jax.experimental import pallas as pl
from jax.experimental.pallas import tpu as pltpu
from jax.experimental.pallas import tpu_sc as plsc

B = 16
T = 16384
D = 128
NR = 5
NR_PAD = 8
IDX_CHUNK = 128
N_CHUNKS = 2
TC_CHUNK = T // N_CHUNKS
TBLK = 8192


def _sc_gather(chunk, flat_users, nodes_v, embed_u_w, embed_i_w):
    info = plsc.get_sparse_core_info()
    nc, ns = info.num_cores, info.num_subcores
    nw = nc * ns
    rows_per_w = TC_CHUNK // nw
    n_idx_chunks = max(rows_per_w // IDX_CHUNK, 1)
    chunk_base = chunk * TC_CHUNK
    with_qj = chunk == 0

    mesh = plsc.VectorSubcoreMesh(core_axis_name="c", subcore_axis_name="s")

    out_type = [jax.ShapeDtypeStruct((TC_CHUNK, D), jnp.float32)]
    scratch = [
        pltpu.VMEM((rows_per_w,), jnp.int32),
        pltpu.VMEM((rows_per_w, D), jnp.float32),
        pltpu.SemaphoreType.DMA,
    ]
    if with_qj:
        out_type.append(jax.ShapeDtypeStruct((B, D), jnp.float32))
        scratch += [
            pltpu.VMEM((B,), jnp.int32),
            pltpu.VMEM((B, D), jnp.float32),
            pltpu.SemaphoreType.DMA,
        ]

    @functools.partial(pl.kernel, mesh=mesh, out_type=out_type,
                       scratch_types=scratch)
    def gather_kernel(*refs):
        if with_qj:
            (users_hbm, nodes_hbm, tab_u, tab_i, out_pt, out_qj,
             idx_v, rows_v, sem, nidx_v, qrows_v, qsem) = refs
        else:
            users_hbm, tab_u, out_pt, idx_v, rows_v, sem = refs
        wid = lax.axis_index("s") * nc + lax.axis_index("c")
        base = wid * rows_per_w
        pltpu.sync_copy(users_hbm.at[pl.ds(chunk_base + base, rows_per_w)], idx_v)
        copies = []
        for c in range(n_idx_chunks):
            copies.append(pltpu.async_copy(
                tab_u.at[idx_v.at[pl.ds(c * IDX_CHUNK, IDX_CHUNK)]],
                rows_v.at[pl.ds(c * IDX_CHUNK, IDX_CHUNK)], sem))
        for cp in copies:
            cp.wait()
        pltpu.sync_copy(rows_v, out_pt.at[pl.ds(base, rows_per_w)])

        if with_qj:
            @pl.when(wid == 0)
            def _():
                pltpu.sync_copy(nodes_hbm, nidx_v)
                pltpu.async_copy(tab_i.at[nidx_v], qrows_v, qsem).wait()
                pltpu.sync_copy(qrows_v, out_qj)

    if with_qj:
        return gather_kernel(flat_users, nodes_v, embed_u_w, embed_i_w)
    return gather_kernel(flat_users, embed_u_w)[0]


_NT = (((1,), (1,)), ((), ()))
_TN = (((0,), (0,)), ((), ()))
_BF = jnp.bfloat16


def _nt_bf16(x, w):
    return lax.dot_general(x.astype(_BF), w.astype(_BF), _NT,
                           preferred_element_type=jnp.float32)


def _tc_body(rat_ref, seg_ref, pt_ref, qj_ref, er_ref, g1_ref, g1b_ref,
             g2_ref, g2b_ref, a1_ref, a1b_ref, a2_ref, a2b_ref, a3_ref,
             *rest, nblk, first, last):
    n_outs = 1 if last else 3
    if first:
        in_state = ()
    else:
        in_state = rest[:3]
        rest = rest[3:]
    outs = rest[:n_outs]
    m_ref, d_ref, z_ref, rtab_ref, stab_ref = rest[n_outs:]

    i = pl.program_id(0)

    @pl.when(i == 0)
    def _init():
        if first:
            m_ref[...] = jnp.full((B, 1), -1e30, jnp.float32)
            d_ref[...] = jnp.zeros((B, 1), jnp.float32)
            z_ref[...] = jnp.zeros((B, D), jnp.float32)
        else:
            m_in, d_in, z_in = in_state
            m_ref[...] = m_in[...]
            d_ref[...] = d_in[...]
            z_ref[...] = z_in[...]
        rtab_ref[:NR, :] = lax.dot_general(er_ref[...], g1_ref[:, D:], _NT)
        rtab_ref[NR:, :] = jnp.zeros((NR_PAD - NR, D), jnp.float32)
        stab_ref[...] = lax.dot_general(qj_ref[...], a1_ref[:, D:], _NT)

    pt = pt_ref[...]
    rat_row = rat_ref[...].reshape(1, TBLK)
    seg_row = seg_ref[...].reshape(1, TBLK)

    rat_ohT = (lax.broadcasted_iota(jnp.int32, (NR_PAD, TBLK), 0)
               == rat_row).astype(jnp.float32)
    seg_ohT = lax.broadcasted_iota(jnp.int32, (B, TBLK), 0) == seg_row

    er_c = lax.dot_general(rat_ohT, rtab_ref[...], _TN)
    h = jnp.maximum(_nt_bf16(pt, g1_ref[:, :D]) + er_c + g1b_ref[...], 0.0)
    fjt = jnp.maximum(_nt_bf16(h, g2_ref[...]) + g2b_ref[...], 0.0)
    seg_c = lax.dot_general(seg_ohT.astype(jnp.float32), stab_ref[...], _TN)
    a = jnp.maximum(_nt_bf16(fjt, a1_ref[:, :D]) + seg_c + a1b_ref[...], 0.0)
    a = jnp.maximum(_nt_bf16(a, a2_ref[...]) + a2b_ref[...], 0.0)
    s_row = lax.dot_general(a3_ref[...], a, _NT)

    neg = jnp.float32(-1e30)
    seg_f = seg_ohT.astype(jnp.float32)
    masked = jnp.where(seg_ohT, s_row, neg)
    blk_m = jnp.max(masked, axis=1, keepdims=True)
    m_old = m_ref[...]
    m_new = jnp.maximum(m_old, blk_m)
    scale = jnp.exp(m_old - m_new)
    m_tok = lax.dot_general(m_new, seg_f, _TN)
    e_row = jnp.exp(s_row - m_tok)
    e_t = seg_f * e_row
    m_ref[...] = m_new
    d_ref[...] = d_ref[...] * scale + jnp.sum(e_t, axis=1, keepdims=True)
    z_ref[...] = (z_ref[...] * scale
                  + lax.dot_general(e_t, fjt, (((1,), (0,)), ((), ()))))

    @pl.when(i == nblk - 1)
    def _finish():
        if last:
            out_ref, = outs
            dd = d_ref[...]
            out_ref[...] = jnp.where(dd > 0, z_ref[...] / dd, 0.0)
        else:
            om_ref, od_ref, oz_ref = outs
            om_ref[...] = m_ref[...]
            od_ref[...] = d_ref[...]
            oz_ref[...] = z_ref[...]


def _tc_chunk(chunk, pt, qj, flat_ratings, segment_ids, embed_r_w, weights,
              state, last):
    (g1_w, g1_b, g2_w, g2_b, a1_w, a1_b, a2_w, a2_b, a3_w) = weights
    nblk = TC_CHUNK // TBLK
    blk0 = chunk * nblk
    first = state is None

    full = lambda shape: pl.BlockSpec(shape, lambda i: tuple(0 for _ in shape))

    in_specs = [
        pl.BlockSpec((TBLK,), lambda i: (blk0 + i,)),
        pl.BlockSpec((TBLK,), lambda i: (blk0 + i,)),
        pl.BlockSpec((TBLK, D), lambda i: (i, 0)),
        full((B, D)),
        full((NR, D)),
        full((D, 2 * D)),
        full((D,)),
        full((D, D)),
        full((D,)),
        full((D, 2 * D)),
        full((D,)),
        full((D, D)),
        full((D,)),
        full((1, D)),
    ]
    args = [flat_ratings, segment_ids, pt, qj, embed_r_w, g1_w, g1_b, g2_w,
            g2_b, a1_w, a1_b, a2_w, a2_b, a3_w]
    if not first:
        in_specs += [full((B, 1)), full((B, 1)), full((B, D))]
        args += list(state)

    if last:
        out_shape = jax.ShapeDtypeStruct((B, D), jnp.float32)
        out_specs = pl.BlockSpec((B, D), lambda i: (0, 0))
    else:
        out_shape = [jax.ShapeDtypeStruct((B, 1), jnp.float32),
                     jax.ShapeDtypeStruct((B, 1), jnp.float32),
                     jax.ShapeDtypeStruct((B, D), jnp.float32)]
        out_specs = [pl.BlockSpec((B, 1), lambda i: (0, 0)),
                     pl.BlockSpec((B, 1), lambda i: (0, 0)),
                     pl.BlockSpec((B, D), lambda i: (0, 0))]

    body = functools.partial(_tc_body, nblk=nblk, first=first, last=last)
    return pl.pallas_call(
        body,
        grid=(nblk,),
        in_specs=in_specs,
        out_specs=out_specs,
        out_shape=out_shape,
        scratch_shapes=[
            pltpu.VMEM((B, 1), jnp.float32),
            pltpu.VMEM((B, 1), jnp.float32),
            pltpu.VMEM((B, D), jnp.float32),
            pltpu.VMEM((NR_PAD, D), jnp.float32),
            pltpu.VMEM((B, D), jnp.float32),
        ],
    )(*args)


def kernel(nodes_v, flat_users, flat_ratings, segment_ids, embed_u_w,
           embed_i_w, embed_r_w, g1_w, g1_b, g2_w, g2_b, a1_w, a1_b,
           a2_w, a2_b, a3_w, a3_b):
    weights = (g1_w, g1_b, g2_w, g2_b, a1_w, a1_b, a2_w, a2_b, a3_w)

    pt0, qj = _sc_gather(0, flat_users, nodes_v, embed_u_w, embed_i_w)
    pts = [pt0] + [_sc_gather(c, flat_users, nodes_v, embed_u_w, embed_i_w)
                   for c in range(1, N_CHUNKS)]

    state = None
    for c in range(N_CHUNKS):
        out = _tc_chunk(c, pts[c], qj, flat_ratings, segment_ids, embed_r_w,
                        weights, state, last=(c == N_CHUNKS - 1))
        state = out
    return out


# speedup vs baseline: 1.0258x; 1.0051x over previous
import functools

import jax
import jax.numpy as jnp
from jax import lax
from jax.experimental import pallas as pl
from jax.experimental.pallas import tpu as pltpu
from jax.experimental.pallas import tpu_sc as plsc

B = 16
T = 16384
D = 128
NR = 5
NR_PAD = 8
IDX_CHUNK = 128
N_CHUNKS = 2
TC_CHUNK = T // N_CHUNKS
TBLK = 8192


def _sc_gather(chunk, flat_users, nodes_v, embed_u_w, embed_i_w):
    info = plsc.get_sparse_core_info()
    nc, ns = info.num_cores, info.num_subcores
    nw = nc * ns
    rows_per_w = TC_CHUNK // nw
    n_idx_chunks = max(rows_per_w // IDX_CHUNK, 1)
    chunk_base = chunk * TC_CHUNK
    with_qj = chunk == 0

    mesh = plsc.VectorSubcoreMesh(core_axis_name="c", subcore_axis_name="s")

    out_type = [jax.ShapeDtypeStruct((TC_CHUNK, D), jnp.float32)]
    scratch = [
        pltpu.VMEM((rows_per_w,), jnp.int32),
        pltpu.VMEM((rows_per_w, D), jnp.float32),
        pltpu.SemaphoreType.DMA,
        pltpu.SemaphoreType.DMA,
    ]
    if with_qj:
        out_type.append(jax.ShapeDtypeStruct((B, D), jnp.float32))
        scratch += [
            pltpu.VMEM((B,), jnp.int32),
            pltpu.VMEM((B, D), jnp.float32),
            pltpu.SemaphoreType.DMA,
        ]

    @functools.partial(pl.kernel, mesh=mesh, out_type=out_type,
                       scratch_types=scratch)
    def gather_kernel(*refs):
        if with_qj:
            (users_hbm, nodes_hbm, tab_u, tab_i, out_pt, out_qj,
             idx_v, rows_v, sem, wsem, nidx_v, qrows_v, qsem) = refs
        else:
            users_hbm, tab_u, out_pt, idx_v, rows_v, sem, wsem = refs
        wid = lax.axis_index("s") * nc + lax.axis_index("c")
        base = wid * rows_per_w
        pltpu.sync_copy(users_hbm.at[pl.ds(chunk_base + base, rows_per_w)], idx_v)
        gathers = []
        for c in range(n_idx_chunks):
            gathers.append(pltpu.async_copy(
                tab_u.at[idx_v.at[pl.ds(c * IDX_CHUNK, IDX_CHUNK)]],
                rows_v.at[pl.ds(c * IDX_CHUNK, IDX_CHUNK)], sem))
        writes = []
        for c in range(n_idx_chunks):
            gathers[c].wait()
            writes.append(pltpu.async_copy(
                rows_v.at[pl.ds(c * IDX_CHUNK, IDX_CHUNK)],
                out_pt.at[pl.ds(base + c * IDX_CHUNK, IDX_CHUNK)], wsem))
        for w in writes:
            w.wait()

        if with_qj:
            @pl.when(wid == 0)
            def _():
                pltpu.sync_copy(nodes_hbm, nidx_v)
                pltpu.async_copy(tab_i.at[nidx_v], qrows_v, qsem).wait()
                pltpu.sync_copy(qrows_v, out_qj)

    if with_qj:
        return gather_kernel(flat_users, nodes_v, embed_u_w, embed_i_w)
    return gather_kernel(flat_users, embed_u_w)[0]


_NT = (((1,), (1,)), ((), ()))
_TN = (((0,), (0,)), ((), ()))
_BF = jnp.bfloat16


def _nt_bf16(x, w):
    return lax.dot_general(x.astype(_BF), w.astype(_BF), _NT,
                           preferred_element_type=jnp.float32)


def _tc_body(rat_ref, seg_ref, pt_ref, qj_ref, er_ref, g1_ref, g1b_ref,
             g2_ref, g2b_ref, a1_ref, a1b_ref, a2_ref, a2b_ref, a3_ref,
             *rest, nblk, first, last):
    n_outs = 1 if last else 3
    if first:
        in_state = ()
    else:
        in_state = rest[:3]
        rest = rest[3:]
    outs = rest[:n_outs]
    m_ref, d_ref, z_ref, rtab_ref, stab_ref = rest[n_outs:]

    i = pl.program_id(0)

    @pl.when(i == 0)
    def _init():
        if first:
            m_ref[...] = jnp.full((B, 1), -1e30, jnp.float32)
            d_ref[...] = jnp.zeros((B, 1), jnp.float32)
            z_ref[...] = jnp.zeros((B, D), jnp.float32)
        else:
            m_in, d_in, z_in = in_state
            m_ref[...] = m_in[...]
            d_ref[...] = d_in[...]
            z_ref[...] = z_in[...]
        rtab_ref[:NR, :] = lax.dot_general(er_ref[...], g1_ref[:, D:], _NT)
        rtab_ref[NR:, :] = jnp.zeros((NR_PAD - NR, D), jnp.float32)
        stab_ref[...] = lax.dot_general(qj_ref[...], a1_ref[:, D:], _NT)

    pt = pt_ref[...]
    rat_row = rat_ref[...].reshape(1, TBLK)
    seg_row = seg_ref[...].reshape(1, TBLK)

    rat_ohT = (lax.broadcasted_iota(jnp.int32, (NR_PAD, TBLK), 0)
               == rat_row).astype(jnp.float32)
    seg_ohT = lax.broadcasted_iota(jnp.int32, (B, TBLK), 0) == seg_row

    er_c = lax.dot_general(rat_ohT, rtab_ref[...], _TN)
    h = jnp.maximum(_nt_bf16(pt, g1_ref[:, :D]) + er_c + g1b_ref[...], 0.0)
    fjt = jnp.maximum(_nt_bf16(h, g2_ref[...]) + g2b_ref[...], 0.0)
    seg_c = lax.dot_general(seg_ohT.astype(jnp.float32), stab_ref[...], _TN)
    a = jnp.maximum(_nt_bf16(fjt, a1_ref[:, :D]) + seg_c + a1b_ref[...], 0.0)
    a = jnp.maximum(_nt_bf16(a, a2_ref[...]) + a2b_ref[...], 0.0)
    s_row = lax.dot_general(a3_ref[...], a, _NT)

    neg = jnp.float32(-1e30)
    seg_f = seg_ohT.astype(jnp.float32)
    masked = jnp.where(seg_ohT, s_row, neg)
    blk_m = jnp.max(masked, axis=1, keepdims=True)
    m_old = m_ref[...]
    m_new = jnp.maximum(m_old, blk_m)
    scale = jnp.exp(m_old - m_new)
    m_tok = lax.dot_general(m_new, seg_f, _TN)
    e_row = jnp.exp(s_row - m_tok)
    e_t = seg_f * e_row
    m_ref[...] = m_new
    d_ref[...] = d_ref[...] * scale + jnp.sum(e_t, axis=1, keepdims=True)
    z_ref[...] = (z_ref[...] * scale
                  + lax.dot_general(e_t, fjt, (((1,), (0,)), ((), ()))))

    @pl.when(i == nblk - 1)
    def _finish():
        if last:
            out_ref, = outs
            dd = d_ref[...]
            out_ref[...] = jnp.where(dd > 0, z_ref[...] / dd, 0.0)
        else:
            om_ref, od_ref, oz_ref = outs
            om_ref[...] = m_ref[...]
            od_ref[...] = d_ref[...]
            oz_ref[...] = z_ref[...]


def _tc_chunk(chunk, pt, qj, flat_ratings, segment_ids, embed_r_w, weights,
              state, last):
    (g1_w, g1_b, g2_w, g2_b, a1_w, a1_b, a2_w, a2_b, a3_w) = weights
    nblk = TC_CHUNK // TBLK
    blk0 = chunk * nblk
    first = state is None

    full = lambda shape: pl.BlockSpec(shape, lambda i: tuple(0 for _ in shape))

    in_specs = [
        pl.BlockSpec((TBLK,), lambda i: (blk0 + i,)),
        pl.BlockSpec((TBLK,), lambda i: (blk0 + i,)),
        pl.BlockSpec((TBLK, D), lambda i: (i, 0)),
        full((B, D)),
        full((NR, D)),
        full((D, 2 * D)),
        full((D,)),
        full((D, D)),
        full((D,)),
        full((D, 2 * D)),
        full((D,)),
        full((D, D)),
        full((D,)),
        full((1, D)),
    ]
    args = [flat_ratings, segment_ids, pt, qj, embed_r_w, g1_w, g1_b, g2_w,
            g2_b, a1_w, a1_b, a2_w, a2_b, a3_w]
    if not first:
        in_specs += [full((B, 1)), full((B, 1)), full((B, D))]
        args += list(state)

    if last:
        out_shape = jax.ShapeDtypeStruct((B, D), jnp.float32)
        out_specs = pl.BlockSpec((B, D), lambda i: (0, 0))
    else:
        out_shape = [jax.ShapeDtypeStruct((B, 1), jnp.float32),
                     jax.ShapeDtypeStruct((B, 1), jnp.float32),
                     jax.ShapeDtypeStruct((B, D), jnp.float32)]
        out_specs = [pl.BlockSpec((B, 1), lambda i: (0, 0)),
                     pl.BlockSpec((B, 1), lambda i: (0, 0)),
                     pl.BlockSpec((B, D), lambda i: (0, 0))]

    body = functools.partial(_tc_body, nblk=nblk, first=first, last=last)
    return pl.pallas_call(
        body,
        grid=(nblk,),
        in_specs=in_specs,
        out_specs=out_specs,
        out_shape=out_shape,
        scratch_shapes=[
            pltpu.VMEM((B, 1), jnp.float32),
            pltpu.VMEM((B, 1), jnp.float32),
            pltpu.VMEM((B, D), jnp.float32),
            pltpu.VMEM((NR_PAD, D), jnp.float32),
            pltpu.VMEM((B, D), jnp.float32),
        ],
    )(*args)


def kernel(nodes_v, flat_users, flat_ratings, segment_ids, embed_u_w,
           embed_i_w, embed_r_w, g1_w, g1_b, g2_w, g2_b, a1_w, a1_b,
           a2_w, a2_b, a3_w, a3_b):
    weights = (g1_w, g1_b, g2_w, g2_b, a1_w, a1_b, a2_w, a2_b, a3_w)

    pt0, qj = _sc_gather(0, flat_users, nodes_v, embed_u_w, embed_i_w)
    pts = [pt0] + [_sc_gather(c, flat_users, nodes_v, embed_u_w, embed_i_w)
                   for c in range(1, N_CHUNKS)]

    state = None
    for c in range(N_CHUNKS):
        out = _tc_chunk(c, pts[c], qj, flat_ratings, segment_ids, embed_r_w,
                        weights, state, last=(c == N_CHUNKS - 1))
        state = out
    return out
